# Initial kernel scaffold; baseline (speedup 1.0000x reference)
import jax
import jax.numpy as jnp
from jax.experimental import pallas as pl


def kernel(pred, target, coords):
    raise NotImplementedError("write your pallas kernel here")


# speedup vs baseline: 65.9443x; 65.9443x over previous
import functools

import jax
import jax.numpy as jnp
from jax import lax
from jax.experimental import pallas as pl
from jax.experimental.pallas import tpu as pltpu
from jax.experimental.pallas import tpu_sc as plsc

N = 8192
RBLK = 256
NUM_BLOCKS = N // RBLK


def _knn_l2_body(c_ref, ct_ref, p_ref, t_ref, idx_ref, l2_ref):
    i = pl.program_id(0)
    rows = c_ref[...]
    g = jnp.dot(rows, ct_ref[...], preferred_element_type=jnp.float32)
    x, y, z = rows[:, 0:1], rows[:, 1:2], rows[:, 2:3]
    a2r = x * x + y * y + z * z
    cx, cy, cz = ct_ref[0:1, :], ct_ref[1:2, :], ct_ref[2:3, :]
    a2c = cx * cx + cy * cy + cz * cz
    d2 = (a2r + a2c) - 2.0 * g
    dist = jnp.sqrt(jnp.maximum(d2, 0.0))
    col_ids = lax.broadcasted_iota(jnp.int32, (RBLK, N), 1)
    big = jnp.float32(3.4e38)
    m1 = jnp.min(dist, axis=1, keepdims=True)
    i1 = jnp.min(jnp.where(dist == m1, col_ids, jnp.int32(N)), axis=1, keepdims=True)
    dist2 = jnp.where(col_ids == i1, big, dist)
    m2 = jnp.min(dist2, axis=1, keepdims=True)
    idx_ref[...] = jnp.min(jnp.where(dist2 == m2, col_ids, jnp.int32(N)), axis=1)

    dpt = p_ref[...] - t_ref[...]
    part = jnp.sum(dpt * dpt)

    @pl.when(i == 0)
    def _():
        l2_ref[0, 0] = 0.0

    l2_ref[0, 0] += part


def _knn_l2(coords, coords_t, pred, target):
    return pl.pallas_call(
        _knn_l2_body,
        grid=(NUM_BLOCKS,),
        in_specs=[
            pl.BlockSpec((RBLK, 3), lambda i: (i, 0)),
            pl.BlockSpec((3, N), lambda i: (0, 0)),
            pl.BlockSpec((RBLK, 3), lambda i: (i, 0)),
            pl.BlockSpec((RBLK, 3), lambda i: (i, 0)),
        ],
        out_specs=[
            pl.BlockSpec((RBLK,), lambda i: (i,)),
            pl.BlockSpec((1, 1), lambda i: (0, 0), memory_space=pltpu.SMEM),
        ],
        out_shape=[
            jax.ShapeDtypeStruct((N,), jnp.int32),
            jax.ShapeDtypeStruct((1, 1), jnp.float32),
        ],
    )(coords, coords_t, pred, target)


_NW = 32
_PER_W = N // _NW
_CHUNK = 128
_NCH = _PER_W // _CHUNK


def _tv_body(pred_hbm, idx_hbm, out_hbm, idx_v, own_v, gat_v, acc_v, sem):
    wid = lax.axis_index("s") * 2 + lax.axis_index("c")
    acc_v[...] = jnp.zeros((16,), jnp.float32)
    for c in range(_NCH):
        base = wid * _PER_W + c * _CHUNK
        pltpu.sync_copy(idx_hbm.at[pl.ds(base, _CHUNK)], idx_v)
        pltpu.async_copy(pred_hbm.at[idx_v], gat_v, sem).wait()
        pltpu.sync_copy(pred_hbm.at[pl.ds(base, _CHUNK)], own_v)

        def body(r, _):
            acc_v[...] = acc_v[...] + jnp.abs(own_v[r, :] - gat_v[r, :])
            return ()

        lax.fori_loop(0, _CHUNK, body, ())
    pltpu.sync_copy(acc_v, out_hbm.at[wid])


@functools.cache
def _tv_partials_fn():
    return pl.kernel(
        _tv_body,
        out_type=jax.ShapeDtypeStruct((_NW, 16), jnp.float32),
        mesh=plsc.VectorSubcoreMesh(core_axis_name="c", subcore_axis_name="s"),
        scratch_types=[
            pltpu.VMEM((_CHUNK,), jnp.int32),
            pltpu.VMEM((_CHUNK, 16), jnp.float32),
            pltpu.VMEM((_CHUNK, 16), jnp.float32),
            pltpu.VMEM((16,), jnp.float32),
            pltpu.SemaphoreType.DMA,
        ],
        compiler_params=pltpu.CompilerParams(use_tc_tiling_on_sc=False),
    )


def kernel(pred, target, coords):
    coords_t = coords.T
    knn_idx, l2_sum = _knn_l2(coords, coords_t, pred, target)
    pred_pad = jnp.pad(pred, ((0, 0), (0, 13)))
    partials = _tv_partials_fn()(pred_pad, knn_idx)
    tv_sum = jnp.sum(partials)
    return l2_sum[0, 0] / jnp.float32(N * 3) + tv_sum / jnp.float32(N)


# speedup vs baseline: 91.2081x; 1.3831x over previous
import functools

import jax
import jax.numpy as jnp
from jax import lax
from jax.experimental import pallas as pl
from jax.experimental.pallas import tpu as pltpu
from jax.experimental.pallas import tpu_sc as plsc

N = 8192
RBLK = 256
NUM_BLOCKS = N // RBLK


def _knn_l2_body(c_ref, ct_ref, ctm2_ref, p_ref, t_ref, idx_ref, l2_ref):
    i = pl.program_id(0)
    rows = c_ref[...]
    g2 = jnp.dot(rows, ctm2_ref[...], preferred_element_type=jnp.float32)
    x, y, z = rows[:, 0:1], rows[:, 1:2], rows[:, 2:3]
    a2r = x * x + y * y + z * z
    cx, cy, cz = ct_ref[0:1, :], ct_ref[1:2, :], ct_ref[2:3, :]
    a2c = cx * cx + cy * cy + cz * cz
    d2 = jnp.maximum((a2r + a2c) + g2, 0.0)
    col_ids = lax.broadcasted_iota(jnp.int32, (RBLK, N), 1)
    big = jnp.float32(3.4e38)
    m1 = jnp.min(d2, axis=1, keepdims=True)
    i1 = jnp.min(jnp.where(d2 == m1, col_ids, jnp.int32(N)), axis=1, keepdims=True)
    d2x = jnp.where(col_ids == i1, big, d2)
    m2 = jnp.min(d2x, axis=1, keepdims=True)
    idx_ref[...] = jnp.min(jnp.where(d2x == m2, col_ids, jnp.int32(N)), axis=1)

    dpt = p_ref[...] - t_ref[...]
    part = jnp.sum(dpt * dpt)

    @pl.when(i == 0)
    def _():
        l2_ref[0, 0] = 0.0

    l2_ref[0, 0] += part


def _knn_l2(coords, coords_t, coords_tm2, pred, target):
    return pl.pallas_call(
        _knn_l2_body,
        grid=(NUM_BLOCKS,),
        in_specs=[
            pl.BlockSpec((RBLK, 3), lambda i: (i, 0)),
            pl.BlockSpec((3, N), lambda i: (0, 0)),
            pl.BlockSpec((3, N), lambda i: (0, 0)),
            pl.BlockSpec((RBLK, 3), lambda i: (i, 0)),
            pl.BlockSpec((RBLK, 3), lambda i: (i, 0)),
        ],
        out_specs=[
            pl.BlockSpec((RBLK,), lambda i: (i,)),
            pl.BlockSpec((1, 1), lambda i: (0, 0), memory_space=pltpu.SMEM),
        ],
        out_shape=[
            jax.ShapeDtypeStruct((N,), jnp.int32),
            jax.ShapeDtypeStruct((1, 1), jnp.float32),
        ],
    )(coords, coords_t, coords_tm2, pred, target)


_NW = 32
_PER_W = N // _NW
_CHUNK = 128
_NCH = _PER_W // _CHUNK


def _tv_body(pred_hbm, idx_hbm, out_hbm, idx_v, own_v, gat_v, acc_v, sem):
    wid = lax.axis_index("s") * 2 + lax.axis_index("c")
    acc_v[...] = jnp.zeros((16,), jnp.float32)
    for c in range(_NCH):
        base = wid * _PER_W + c * _CHUNK
        pltpu.sync_copy(idx_hbm.at[pl.ds(base, _CHUNK)], idx_v)
        pltpu.async_copy(pred_hbm.at[idx_v], gat_v, sem).wait()
        pltpu.sync_copy(pred_hbm.at[pl.ds(base, _CHUNK)], own_v)

        def body(r, _):
            acc_v[...] = acc_v[...] + jnp.abs(own_v[r, :] - gat_v[r, :])
            return ()

        lax.fori_loop(0, _CHUNK, body, ())
    pltpu.sync_copy(acc_v, out_hbm.at[wid])


@functools.cache
def _tv_partials_fn():
    return pl.kernel(
        _tv_body,
        out_type=jax.ShapeDtypeStruct((_NW, 16), jnp.float32),
        mesh=plsc.VectorSubcoreMesh(core_axis_name="c", subcore_axis_name="s"),
        scratch_types=[
            pltpu.VMEM((_CHUNK,), jnp.int32),
            pltpu.VMEM((_CHUNK, 16), jnp.float32),
            pltpu.VMEM((_CHUNK, 16), jnp.float32),
            pltpu.VMEM((16,), jnp.float32),
            pltpu.SemaphoreType.DMA,
        ],
        compiler_params=pltpu.CompilerParams(use_tc_tiling_on_sc=False),
    )


def kernel(pred, target, coords):
    coords_t = coords.T
    knn_idx, l2_sum = _knn_l2(coords, coords_t, coords_t * jnp.float32(-2.0), pred, target)
    pred_pad = jnp.pad(pred, ((0, 0), (0, 13)))
    partials = _tv_partials_fn()(pred_pad, knn_idx)
    tv_sum = jnp.sum(partials)
    return l2_sum[0, 0] / jnp.float32(N * 3) + tv_sum / jnp.float32(N)


# speedup vs baseline: 103.5988x; 1.1359x over previous
import functools

import jax
import jax.numpy as jnp
from jax import lax
from jax.experimental import pallas as pl
from jax.experimental.pallas import tpu as pltpu
from jax.experimental.pallas import tpu_sc as plsc

N = 8192
RBLK = 256
NUM_BLOCKS = N // RBLK


def _knn_l2_body(c_ref, ctm2_ref, p_ref, t_ref, idx_ref, l2_ref, pad_ref):
    i = pl.program_id(0)
    rows = c_ref[...]
    g2 = jnp.dot(rows, ctm2_ref[...], preferred_element_type=jnp.float32)
    x, y, z = rows[:, 0:1], rows[:, 1:2], rows[:, 2:3]
    a2r = x * x + y * y + z * z
    u, v, w = ctm2_ref[0:1, :], ctm2_ref[1:2, :], ctm2_ref[2:3, :]
    a2c = (u * u + v * v + w * w) * 0.25
    d2 = jnp.maximum((a2r + a2c) + g2, 0.0)
    colf = lax.broadcasted_iota(jnp.int32, (RBLK, N), 1).astype(jnp.float32)
    big = jnp.float32(3.4e38)
    sent = jnp.float32(16384.0)
    m1 = jnp.min(d2, axis=1, keepdims=True)
    i1 = jnp.min(jnp.where(d2 == m1, colf, sent), axis=1, keepdims=True)
    d2x = jnp.where(colf == i1, big, d2)
    m2 = jnp.min(d2x, axis=1, keepdims=True)
    i2 = jnp.min(jnp.where(d2x == m2, colf, sent), axis=1)
    idx_ref[...] = i2.astype(jnp.int32)

    p = p_ref[...]
    dpt = p - t_ref[...]
    part = jnp.sum(dpt * dpt)

    @pl.when(i == 0)
    def _():
        l2_ref[0, 0] = 0.0

    l2_ref[0, 0] += part

    pad_ref[...] = jnp.pad(p, ((0, 0), (0, 13)))


def _knn_l2(coords, coords_tm2, pred, target):
    return pl.pallas_call(
        _knn_l2_body,
        grid=(NUM_BLOCKS,),
        in_specs=[
            pl.BlockSpec((RBLK, 3), lambda i: (i, 0)),
            pl.BlockSpec((3, N), lambda i: (0, 0)),
            pl.BlockSpec((RBLK, 3), lambda i: (i, 0)),
            pl.BlockSpec((RBLK, 3), lambda i: (i, 0)),
        ],
        out_specs=[
            pl.BlockSpec((RBLK,), lambda i: (i,)),
            pl.BlockSpec((1, 1), lambda i: (0, 0), memory_space=pltpu.SMEM),
            pl.BlockSpec((RBLK, 16), lambda i: (i, 0)),
        ],
        out_shape=[
            jax.ShapeDtypeStruct((N,), jnp.int32),
            jax.ShapeDtypeStruct((1, 1), jnp.float32),
            jax.ShapeDtypeStruct((N, 16), jnp.float32),
        ],
    )(coords, coords_tm2, pred, target)


_NW = 32
_PER_W = N // _NW
_CHUNK = 128
_NCH = _PER_W // _CHUNK


def _tv_body(pred_hbm, idx_hbm, out_hbm, idx_v, own_v, gat_v, acc_v, sem):
    wid = lax.axis_index("s") * 2 + lax.axis_index("c")
    acc_v[...] = jnp.zeros((16,), jnp.float32)
    for c in range(_NCH):
        base = wid * _PER_W + c * _CHUNK
        pltpu.sync_copy(idx_hbm.at[pl.ds(base, _CHUNK)], idx_v)
        pltpu.async_copy(pred_hbm.at[idx_v], gat_v, sem).wait()
        pltpu.sync_copy(pred_hbm.at[pl.ds(base, _CHUNK)], own_v)

        def body(r, _):
            acc_v[...] = acc_v[...] + jnp.abs(own_v[r, :] - gat_v[r, :])
            return ()

        lax.fori_loop(0, _CHUNK, body, ())
    pltpu.sync_copy(acc_v, out_hbm.at[wid])


@functools.cache
def _tv_partials_fn():
    return pl.kernel(
        _tv_body,
        out_type=jax.ShapeDtypeStruct((_NW, 16), jnp.float32),
        mesh=plsc.VectorSubcoreMesh(core_axis_name="c", subcore_axis_name="s"),
        scratch_types=[
            pltpu.VMEM((_CHUNK,), jnp.int32),
            pltpu.VMEM((_CHUNK, 16), jnp.float32),
            pltpu.VMEM((_CHUNK, 16), jnp.float32),
            pltpu.VMEM((16,), jnp.float32),
            pltpu.SemaphoreType.DMA,
        ],
        compiler_params=pltpu.CompilerParams(use_tc_tiling_on_sc=False),
    )


def kernel(pred, target, coords):
    coords_tm2 = (coords * jnp.float32(-2.0)).T
    knn_idx, l2_sum, pred_pad = _knn_l2(coords, coords_tm2, pred, target)
    partials = _tv_partials_fn()(pred_pad, knn_idx)
    tv_sum = jnp.sum(partials)
    return l2_sum[0, 0] / jnp.float32(N * 3) + tv_sum / jnp.float32(N)
